# Initial kernel scaffold; baseline (speedup 1.0000x reference)
import jax
import jax.numpy as jnp
from jax.experimental import pallas as pl


def kernel(h, aug_D_0, aug_D_1, aug_A_0, aug_A_1, edge_index_mp0, edge_index_mp1, Wd, bd, Wa, ba, params0, params1):
    raise NotImplementedError("write your pallas kernel here")


# speedup vs baseline: 26.6097x; 26.6097x over previous
import jax
import jax.numpy as jnp
from jax import lax
from jax.experimental import pallas as pl
from jax.experimental.pallas import tpu as pltpu
from jax.experimental.pallas import tpu_sc as plsc

N = 10000
E = 160000
HEADS = 4
HID = 64
DOUT = HEADS * HID
SEMD = 128
LABELS = 8

NC = 2
NS = 16
L = 16

EPT = E // NS
CH = 80
NCHUNK = EPT // CH
VPC = CH // L
NDN = 10240
ROWS_PT = N // NS

BN = 1000
GRID = N // BN


def _prep_body(h_ref, d0, d1, a0, a1, wd, bdr, wa, bar,
               w00, w01, w10, w11, as0, as1, as2, as3,
               F0, F1, F2, F3, E0, E1, E2, E3):
    mD = (d0[...] + d1[...]) * 0.5
    dD = jnp.dot(mD, wd[...], preferred_element_type=jnp.float32) + bdr[...]
    mA = (a0[...] + a1[...]) * 0.5
    dA = jnp.dot(mA, wa[...], preferred_element_type=jnp.float32) + bar[...]
    h1 = jnp.concatenate([dD, dA], axis=1)
    hh = h_ref[...]
    for F_out, E_out, x, W, As in ((F0, E0, hh, w00, as0),
                                   (F1, E1, hh, w01, as1),
                                   (F2, E2, h1, w10, as2),
                                   (F3, E3, h1, w11, as3)):
        F = jnp.dot(x, W[...], preferred_element_type=jnp.float32)
        for hd in range(HEADS):
            F_out[hd] = F[:, hd * HID:(hd + 1) * HID]
        E_out[...] = jnp.dot(F, As[...], preferred_element_type=jnp.float32)


def _run_prep(h, aD0, aD1, aA0, aA1, Wd, bd, Wa, ba, Ws_list, As_list):
    full = lambda shape: pl.BlockSpec(shape, lambda i: (0,) * len(shape))
    row = lambda shape: pl.BlockSpec(shape, lambda i: (i,) + (0,) * (len(shape) - 1))
    fout = pl.BlockSpec((HEADS, BN, HID), lambda i: (0, i, 0))
    in_specs = [row((BN, 128))] + [row((BN, 64))] * 4 + \
        [full((64, 64)), full((1, 64)), full((64, 64)), full((1, 64))] + \
        [full((128, DOUT))] * 4 + [full((DOUT, 8))] * 4
    out_specs = [fout] * 4 + [row((BN, 8))] * 4
    out_shape = [jax.ShapeDtypeStruct((HEADS, N, HID), jnp.float32)] * 4 + \
                [jax.ShapeDtypeStruct((N, 8), jnp.float32)] * 4
    return pl.pallas_call(
        _prep_body,
        grid=(GRID,),
        in_specs=in_specs,
        out_specs=out_specs,
        out_shape=out_shape,
    )(h, aD0, aD1, aA0, aA1, Wd, bd.reshape(1, 64), Wa, ba.reshape(1, 64),
      *Ws_list, *As_list)


def _gat_sc_body(srcr, dstr, elr, err, f0, f1, f2, f3, outr,
                 src2, dst2, el_v, er_v, exC, dn, rows, zdn,
                 acc, sdn):
    c = lax.axis_index("c")
    s = lax.axis_index("s")
    zero16 = jnp.zeros((L,), jnp.float32)

    pltpu.sync_copy(srcr.at[pl.ds(s * NCHUNK, NCHUNK)], src2)
    pltpu.sync_copy(dstr.at[pl.ds(s * NCHUNK, NCHUNK)], dst2)
    pltpu.sync_copy(elr.at[pl.ds(2 * c, 2)], el_v)
    pltpu.sync_copy(err.at[pl.ds(2 * c, 2)], er_v)

    def _zrow(r, carry):
        for k in range(HID // L):
            rows[r, pl.ds(k * L, L)] = zero16
        return carry

    r0 = s * ROWS_PT

    def _zero_acc_dn():
        lax.fori_loop(0, CH, _zrow, 0)
        for q in range(ROWS_PT // CH):
            pltpu.sync_copy(rows, acc.at[pl.ds(r0 + q * CH, CH)])
        rem = ROWS_PT % CH
        pltpu.sync_copy(rows.at[pl.ds(0, rem)],
                        acc.at[pl.ds(r0 + ROWS_PT - rem, rem)])

        def _zdn(i, carry):
            zdn[pl.ds(i * L, L)] = zero16
            return carry
        lax.fori_loop(0, (NDN // NS) // L, _zdn, 0)
        pltpu.sync_copy(zdn, sdn.at[pl.ds(s * (NDN // NS), NDN // NS)])

    _zero_acc_dn()
    plsc.subcore_barrier()

    for hl in range(2):
        hvec = jnp.full((L,), hl, jnp.int32)

        def _phaseA(j, carry):
            for m in range(VPC):
                sv = src2[j, pl.ds(m * L, L)]
                dv = dst2[j, pl.ds(m * L, L)]
                elg = plsc.load_gather(el_v, [hvec, sv])
                erg = plsc.load_gather(er_v, [hvec, dv])
                e = elg + erg
                e = jnp.where(e >= 0.0, e, 0.2 * e)
                exC[pl.ds(j * CH + m * L, L)] = jnp.exp(e)
            return carry
        lax.fori_loop(0, NCHUNK, _phaseA, 0)

        def _dstream(j, carry):
            pltpu.sync_copy(exC.at[pl.ds(j * CH, CH)],
                            sdn.at[dst2.at[j]], add=True)
            return carry
        lax.fori_loop(0, NCHUNK, _dstream, 0)

        plsc.subcore_barrier()

        pltpu.sync_copy(sdn, dn)

        def _phaseB(j, carry):
            for m in range(VPC):
                dv = dst2[j, pl.ds(m * L, L)]
                ex = exC[pl.ds(j * CH + m * L, L)]
                dnv = plsc.load_gather(dn, [dv])
                exC[pl.ds(j * CH + m * L, L)] = ex / (dnv + 1e-9)
            return carry
        lax.fori_loop(0, NCHUNK, _phaseB, 0)

        def _phaseC_one(f):
            def _chunk(j, carry):
                pltpu.sync_copy(f.at[src2.at[j]], rows)

                def _scale(m, inner):
                    av = exC[pl.ds(j * CH + m * L, L)]
                    for r16 in range(L):
                        r = m * L + r16
                        a = av[r16]
                        for k in range(HID // L):
                            rows[r, pl.ds(k * L, L)] = (
                                rows[r, pl.ds(k * L, L)] * a)
                    return inner
                lax.fori_loop(0, VPC, _scale, 0)
                pltpu.sync_copy(rows, acc.at[dst2.at[j]], add=True)
                return carry
            lax.fori_loop(0, NCHUNK, _chunk, 0)

        fa, fb = (f0, f2) if hl == 0 else (f1, f3)

        @pl.when(c == 0)
        def _():
            _phaseC_one(fa)

        @pl.when(c == 1)
        def _():
            _phaseC_one(fb)

        plsc.subcore_barrier()

        pltpu.sync_copy(
            acc.at[pl.ds(s * ROWS_PT, ROWS_PT)],
            outr.at[pl.ds((2 * c + hl) * N + s * ROWS_PT, ROWS_PT)])

        if hl == 0:
            _zero_acc_dn()
            plsc.subcore_barrier()


_gat_sc = pl.kernel(
    _gat_sc_body,
    out_type=jax.ShapeDtypeStruct((HEADS * N, HID), jnp.float32),
    mesh=plsc.VectorSubcoreMesh(core_axis_name="c", subcore_axis_name="s"),
    scratch_types=[
        pltpu.VMEM((NCHUNK, CH), jnp.int32),
        pltpu.VMEM((NCHUNK, CH), jnp.int32),
        pltpu.VMEM((2, N), jnp.float32),
        pltpu.VMEM((2, N), jnp.float32),
        pltpu.VMEM((EPT,), jnp.float32),
        pltpu.VMEM((NDN,), jnp.float32),
        pltpu.VMEM((CH, HID), jnp.float32),
        pltpu.VMEM((NDN // NS,), jnp.float32),
        pltpu.VMEM_SHARED((N, HID), jnp.float32),
        pltpu.VMEM_SHARED((NDN,), jnp.float32),
    ],
    compiler_params=pltpu.CompilerParams(use_tc_tiling_on_sc=False,
                                         needs_layout_passes=False),
)


def _assemble(ar):
    z = jnp.concatenate([ar[hd] for hd in range(HEADS)], axis=1)
    return jnp.where(z > 0, z, jnp.exp(z) - 1.0)


def _sem_body(a0r, a1r, a2r, a3r, ws0, bs0, qs0, ws1, bs1, qs1, wref):
    i = pl.program_id(0)
    parts = []
    for ar, ws, bs, qs in ((a0r, ws0, bs0, qs0), (a1r, ws0, bs0, qs0),
                           (a2r, ws1, bs1, qs1), (a3r, ws1, bs1, qs1)):
        z = _assemble(ar)
        t = jnp.tanh(jnp.dot(z, ws[...], preferred_element_type=jnp.float32)
                     + bs[...])
        u = jnp.dot(t, qs[...], preferred_element_type=jnp.float32)
        parts.append(jnp.sum(u).reshape(1, 1))
    wvec = jnp.concatenate(parts, axis=1)

    @pl.when(i == 0)
    def _():
        wref[...] = wvec

    @pl.when(i != 0)
    def _():
        wref[...] = wref[...] + wvec


def _run_sem(aggs, p0, p1):
    full = lambda shape: pl.BlockSpec(shape, lambda i: (0,) * len(shape))
    ain = pl.BlockSpec((HEADS, BN, HID), lambda i: (0, i, 0))
    return pl.pallas_call(
        _sem_body,
        grid=(GRID,),
        in_specs=[ain] * 4 + [full((DOUT, SEMD)), full((1, SEMD)),
                              full((SEMD, 1))] * 2,
        out_specs=pl.BlockSpec((1, 4), lambda i: (0, 0)),
        out_shape=jax.ShapeDtypeStruct((1, 4), jnp.float32),
    )(*aggs, p0['Ws'], p0['bs'].reshape(1, SEMD), p0['qs'].reshape(SEMD, 1),
      p1['Ws'], p1['bs'].reshape(1, SEMD), p1['qs'].reshape(SEMD, 1))


def _head_body(a0r, a1r, a2r, a3r, br, wp0, bp0, wp1, bp1, outr):
    b = br[...]
    z0 = _assemble(a0r)
    z1 = _assemble(a1r)
    z2 = _assemble(a2r)
    z3 = _assemble(a3r)
    s0 = z0 * b[0:1, 0:1] + z1 * b[0:1, 1:2]
    l0 = jnp.dot(s0, wp0[...], preferred_element_type=jnp.float32) + bp0[...]
    s1 = z2 * b[0:1, 2:3] + z3 * b[0:1, 3:4]
    l1 = jnp.dot(s1, wp1[...], preferred_element_type=jnp.float32) + bp1[...]
    outr[...] = jax.nn.sigmoid(l0 + 0.1 * l1)


def _run_head(aggs, beta, p0, p1):
    full = lambda shape: pl.BlockSpec(shape, lambda i: (0,) * len(shape))
    ain = pl.BlockSpec((HEADS, BN, HID), lambda i: (0, i, 0))
    return pl.pallas_call(
        _head_body,
        grid=(GRID,),
        in_specs=[ain] * 4 + [full((1, 4)), full((DOUT, LABELS)),
                              full((1, LABELS)), full((DOUT, LABELS)),
                              full((1, LABELS))],
        out_specs=pl.BlockSpec((BN, LABELS), lambda i: (i, 0)),
        out_shape=jax.ShapeDtypeStruct((N, LABELS), jnp.float32),
    )(*aggs, beta, p0['Wp'], p0['bp'].reshape(1, LABELS),
      p1['Wp'], p1['bp'].reshape(1, LABELS))


def _attn_select(p, m):
    al = p['al%d' % m]
    ar = p['ar%d' % m]
    A = jnp.zeros((DOUT, 8), jnp.float32)
    for hd in range(HEADS):
        A = A.at[hd * HID:(hd + 1) * HID, hd].set(al[hd])
        A = A.at[hd * HID:(hd + 1) * HID, 4 + hd].set(ar[hd])
    return A


def kernel(h, aug_D_0, aug_D_1, aug_A_0, aug_A_1, edge_index_mp0,
           edge_index_mp1, Wd, bd, Wa, ba, params0, params1):
    As_list = [_attn_select(params0, 0), _attn_select(params0, 1),
               _attn_select(params1, 0), _attn_select(params1, 1)]
    Ws_list = [params0['W0'], params0['W1'], params1['W0'], params1['W1']]

    prep = _run_prep(h, aug_D_0, aug_D_1, aug_A_0, aug_A_1,
                     Wd, bd, Wa, ba, Ws_list, As_list)
    feats = prep[:4]
    elers = prep[4:]

    edges = []
    for ei in (edge_index_mp0, edge_index_mp1):
        src = ei[0].reshape(NS * NCHUNK, CH)
        dst = ei[1].reshape(NS * NCHUNK, CH)
        edges.append((src, dst))

    aggs = []
    for li, (F, eler) in enumerate(zip(feats, elers)):
        src, dst = edges[li % 2]
        elerT = jnp.transpose(eler)
        elT = elerT[:HEADS]
        erT = elerT[HEADS:]
        out = _gat_sc(src, dst, elT, erT, F[0], F[1], F[2], F[3])
        aggs.append(out.reshape(HEADS, N, HID))

    wsum = _run_sem(aggs, params0, params1)
    w = wsum[0] / float(N)
    beta = jnp.concatenate([jax.nn.softmax(w[:2]), jax.nn.softmax(w[2:])])
    beta = beta.reshape(1, 4)

    return _run_head(aggs, beta, params0, params1)


# speedup vs baseline: 35.7607x; 1.3439x over previous
import jax
import jax.numpy as jnp
import numpy as np
from jax import lax
from jax.experimental import pallas as pl
from jax.experimental.pallas import tpu as pltpu
from jax.experimental.pallas import tpu_sc as plsc

N = 10000
E = 160000
HEADS = 4
HID = 64
DOUT = HEADS * HID
SEMD = 128
LABELS = 8

NC = 2
NS = 16
L = 16

EPT = E // NS
CH = 80
NCHUNK = EPT // CH
VPC = CH // L
NDN = 10240
ROWS_PT = N // NS

BN = 1000
GRID = N // BN

MASKHI = -65536
PACK_PERM = tuple(c for w in range(HID // 2) for c in (w, w + HID // 2))


def _prep_body(h_ref, d0, d1, a0, a1, wd, bdr, wa, bar,
               w00, w01, w10, w11, as0, as1, as2, as3,
               F0, F1, F2, F3, E0, E1, E2, E3):
    mD = (d0[...] + d1[...]) * 0.5
    dD = jnp.dot(mD, wd[...], preferred_element_type=jnp.float32) + bdr[...]
    mA = (a0[...] + a1[...]) * 0.5
    dA = jnp.dot(mA, wa[...], preferred_element_type=jnp.float32) + bar[...]
    h1 = jnp.concatenate([dD, dA], axis=1)
    hh = h_ref[...]
    for F_out, E_out, x, W, As in ((F0, E0, hh, w00, as0),
                                   (F1, E1, hh, w01, as1),
                                   (F2, E2, h1, w10, as2),
                                   (F3, E3, h1, w11, as3)):
        F = jnp.dot(x, W[...], preferred_element_type=jnp.float32)
        for hd in range(HEADS):
            F_out[hd] = F[:, hd * HID:(hd + 1) * HID]
        E_out[...] = jnp.dot(F, As[...], preferred_element_type=jnp.float32)


def _run_prep(h, aD0, aD1, aA0, aA1, Wd, bd, Wa, ba, Ws_list, As_list):
    full = lambda shape: pl.BlockSpec(shape, lambda i: (0,) * len(shape))
    row = lambda shape: pl.BlockSpec(shape, lambda i: (i,) + (0,) * (len(shape) - 1))
    fout = pl.BlockSpec((HEADS, BN, HID), lambda i: (0, i, 0))
    in_specs = [row((BN, 128))] + [row((BN, 64))] * 4 + \
        [full((64, 64)), full((1, 64)), full((64, 64)), full((1, 64))] + \
        [full((128, DOUT))] * 4 + [full((DOUT, 8))] * 4
    out_specs = [fout] * 4 + [row((BN, 8))] * 4
    out_shape = [jax.ShapeDtypeStruct((HEADS, N, HID), jnp.float32)] * 4 + \
                [jax.ShapeDtypeStruct((N, 8), jnp.float32)] * 4
    return pl.pallas_call(
        _prep_body,
        grid=(GRID,),
        in_specs=in_specs,
        out_specs=out_specs,
        out_shape=out_shape,
    )(h, aD0, aD1, aA0, aA1, Wd, bd.reshape(1, 64), Wa, ba.reshape(1, 64),
      *Ws_list, *As_list)


def _gat_sc_body(srcr, dstr, elr, err, f0, f1, f2, f3, outr,
                 src2, dst2, el_v, er_v, exC, dn, rows, rows_i, zdn,
                 acc, sdn):
    c = lax.axis_index("c")
    s = lax.axis_index("s")
    zero16 = jnp.zeros((L,), jnp.float32)

    pltpu.sync_copy(srcr.at[pl.ds(s * NCHUNK, NCHUNK)], src2)
    pltpu.sync_copy(dstr.at[pl.ds(s * NCHUNK, NCHUNK)], dst2)
    pltpu.sync_copy(elr.at[pl.ds(2 * c, 2)], el_v)
    pltpu.sync_copy(err.at[pl.ds(2 * c, 2)], er_v)

    def _zrow(r, carry):
        for k in range(HID // L):
            rows[r, pl.ds(k * L, L)] = zero16
        return carry

    r0 = s * ROWS_PT

    def _zero_acc_dn():
        lax.fori_loop(0, CH, _zrow, 0)
        for q in range(ROWS_PT // CH):
            pltpu.sync_copy(rows, acc.at[pl.ds(r0 + q * CH, CH)])
        rem = ROWS_PT % CH
        pltpu.sync_copy(rows.at[pl.ds(0, rem)],
                        acc.at[pl.ds(r0 + ROWS_PT - rem, rem)])

        def _zdn(i, carry):
            zdn[pl.ds(i * L, L)] = zero16
            return carry
        lax.fori_loop(0, (NDN // NS) // L, _zdn, 0)
        pltpu.sync_copy(zdn, sdn.at[pl.ds(s * (NDN // NS), NDN // NS)])

    _zero_acc_dn()
    plsc.subcore_barrier()

    for hl in range(2):
        hvec = jnp.full((L,), hl, jnp.int32)

        def _phaseA(j, carry):
            for m in range(VPC):
                sv = src2[j, pl.ds(m * L, L)]
                dv = dst2[j, pl.ds(m * L, L)]
                elg = plsc.load_gather(el_v, [hvec, sv])
                erg = plsc.load_gather(er_v, [hvec, dv])
                e = elg + erg
                e = jnp.where(e >= 0.0, e, 0.2 * e)
                exC[pl.ds(j * CH + m * L, L)] = jnp.exp(e)
            return carry
        lax.fori_loop(0, NCHUNK, _phaseA, 0)

        def _dstream(j, carry):
            pltpu.sync_copy(exC.at[pl.ds(j * CH, CH)],
                            sdn.at[dst2.at[j]], add=True)
            return carry
        lax.fori_loop(0, NCHUNK, _dstream, 0)

        plsc.subcore_barrier()

        pltpu.sync_copy(sdn, dn)

        def _phaseB(j, carry):
            for m in range(VPC):
                dv = dst2[j, pl.ds(m * L, L)]
                ex = exC[pl.ds(j * CH + m * L, L)]
                dnv = plsc.load_gather(dn, [dv])
                exC[pl.ds(j * CH + m * L, L)] = ex / (dnv + 1e-9)
            return carry
        lax.fori_loop(0, NCHUNK, _phaseB, 0)

        def _phaseC_one(f):
            def _chunk(j, carry):
                pltpu.sync_copy(f.at[src2.at[j]], rows_i)

                def _scale(m, inner):
                    av = exC[pl.ds(j * CH + m * L, L)]
                    for r16 in range(L):
                        r = m * L + r16
                        a = av[r16]
                        w0 = rows_i[r, pl.ds(0, L)]
                        w1 = rows_i[r, pl.ds(L, L)]
                        rows[r, pl.ds(0, L)] = lax.bitcast_convert_type(
                            jnp.left_shift(w0, 16), jnp.float32) * a
                        rows[r, pl.ds(2 * L, L)] = lax.bitcast_convert_type(
                            jnp.bitwise_and(w0, MASKHI), jnp.float32) * a
                        rows[r, pl.ds(L, L)] = lax.bitcast_convert_type(
                            jnp.left_shift(w1, 16), jnp.float32) * a
                        rows[r, pl.ds(3 * L, L)] = lax.bitcast_convert_type(
                            jnp.bitwise_and(w1, MASKHI), jnp.float32) * a
                    return inner
                lax.fori_loop(0, VPC, _scale, 0)
                pltpu.sync_copy(rows, acc.at[dst2.at[j]], add=True)
                return carry
            lax.fori_loop(0, NCHUNK, _chunk, 0)

        fa, fb = (f0, f2) if hl == 0 else (f1, f3)

        @pl.when(c == 0)
        def _():
            _phaseC_one(fa)

        @pl.when(c == 1)
        def _():
            _phaseC_one(fb)

        plsc.subcore_barrier()

        pltpu.sync_copy(
            acc.at[pl.ds(s * ROWS_PT, ROWS_PT)],
            outr.at[pl.ds((2 * c + hl) * N + s * ROWS_PT, ROWS_PT)])

        if hl == 0:
            _zero_acc_dn()
            plsc.subcore_barrier()


_gat_sc = pl.kernel(
    _gat_sc_body,
    out_type=jax.ShapeDtypeStruct((HEADS * N, HID), jnp.float32),
    mesh=plsc.VectorSubcoreMesh(core_axis_name="c", subcore_axis_name="s"),
    scratch_types=[
        pltpu.VMEM((NCHUNK, CH), jnp.int32),
        pltpu.VMEM((NCHUNK, CH), jnp.int32),
        pltpu.VMEM((2, N), jnp.float32),
        pltpu.VMEM((2, N), jnp.float32),
        pltpu.VMEM((EPT,), jnp.float32),
        pltpu.VMEM((NDN,), jnp.float32),
        pltpu.VMEM((CH, HID), jnp.float32),
        pltpu.VMEM((CH, HID // 2), jnp.int32),
        pltpu.VMEM((NDN // NS,), jnp.float32),
        pltpu.VMEM_SHARED((N, HID), jnp.float32),
        pltpu.VMEM_SHARED((NDN,), jnp.float32),
    ],
    compiler_params=pltpu.CompilerParams(use_tc_tiling_on_sc=False,
                                         needs_layout_passes=False),
)


def _assemble(ar):
    z = jnp.concatenate([ar[hd] for hd in range(HEADS)], axis=1)
    return jnp.where(z > 0, z, jnp.exp(z) - 1.0)


def _sem_body(a0r, a1r, a2r, a3r, ws0, bs0, qs0, ws1, bs1, qs1, wref):
    i = pl.program_id(0)
    parts = []
    for ar, ws, bs, qs in ((a0r, ws0, bs0, qs0), (a1r, ws0, bs0, qs0),
                           (a2r, ws1, bs1, qs1), (a3r, ws1, bs1, qs1)):
        z = _assemble(ar)
        t = jnp.tanh(jnp.dot(z, ws[...], preferred_element_type=jnp.float32)
                     + bs[...])
        u = jnp.dot(t, qs[...], preferred_element_type=jnp.float32)
        parts.append(jnp.sum(u).reshape(1, 1))
    wvec = jnp.concatenate(parts, axis=1)

    @pl.when(i == 0)
    def _():
        wref[...] = wvec

    @pl.when(i != 0)
    def _():
        wref[...] = wref[...] + wvec


def _run_sem(aggs, p0, p1):
    full = lambda shape: pl.BlockSpec(shape, lambda i: (0,) * len(shape))
    ain = pl.BlockSpec((HEADS, BN, HID), lambda i: (0, i, 0))
    return pl.pallas_call(
        _sem_body,
        grid=(GRID,),
        in_specs=[ain] * 4 + [full((DOUT, SEMD)), full((1, SEMD)),
                              full((SEMD, 1))] * 2,
        out_specs=pl.BlockSpec((1, 4), lambda i: (0, 0)),
        out_shape=jax.ShapeDtypeStruct((1, 4), jnp.float32),
    )(*aggs, p0['Ws'], p0['bs'].reshape(1, SEMD), p0['qs'].reshape(SEMD, 1),
      p1['Ws'], p1['bs'].reshape(1, SEMD), p1['qs'].reshape(SEMD, 1))


def _head_body(a0r, a1r, a2r, a3r, br, wp0, bp0, wp1, bp1, outr):
    b = br[...]
    z0 = _assemble(a0r)
    z1 = _assemble(a1r)
    z2 = _assemble(a2r)
    z3 = _assemble(a3r)
    s0 = z0 * b[0:1, 0:1] + z1 * b[0:1, 1:2]
    l0 = jnp.dot(s0, wp0[...], preferred_element_type=jnp.float32) + bp0[...]
    s1 = z2 * b[0:1, 2:3] + z3 * b[0:1, 3:4]
    l1 = jnp.dot(s1, wp1[...], preferred_element_type=jnp.float32) + bp1[...]
    outr[...] = jax.nn.sigmoid(l0 + 0.1 * l1)


def _run_head(aggs, beta, p0, p1):
    full = lambda shape: pl.BlockSpec(shape, lambda i: (0,) * len(shape))
    ain = pl.BlockSpec((HEADS, BN, HID), lambda i: (0, i, 0))
    return pl.pallas_call(
        _head_body,
        grid=(GRID,),
        in_specs=[ain] * 4 + [full((1, 4)), full((DOUT, LABELS)),
                              full((1, LABELS)), full((DOUT, LABELS)),
                              full((1, LABELS))],
        out_specs=pl.BlockSpec((BN, LABELS), lambda i: (i, 0)),
        out_shape=jax.ShapeDtypeStruct((N, LABELS), jnp.float32),
    )(*aggs, beta, p0['Wp'], p0['bp'].reshape(1, LABELS),
      p1['Wp'], p1['bp'].reshape(1, LABELS))


def _attn_select(p, m):
    al = p['al%d' % m]
    ar = p['ar%d' % m]
    A = jnp.zeros((DOUT, 8), jnp.float32)
    for hd in range(HEADS):
        A = A.at[hd * HID:(hd + 1) * HID, hd].set(al[hd])
        A = A.at[hd * HID:(hd + 1) * HID, 4 + hd].set(ar[hd])
    return A


def kernel(h, aug_D_0, aug_D_1, aug_A_0, aug_A_1, edge_index_mp0,
           edge_index_mp1, Wd, bd, Wa, ba, params0, params1):
    As_list = [_attn_select(params0, 0), _attn_select(params0, 1),
               _attn_select(params1, 0), _attn_select(params1, 1)]
    Ws_list = [params0['W0'], params0['W1'], params1['W0'], params1['W1']]

    prep = _run_prep(h, aug_D_0, aug_D_1, aug_A_0, aug_A_1,
                     Wd, bd, Wa, ba, Ws_list, As_list)
    feats = prep[:4]
    elers = prep[4:]

    edges = []
    for ei in (edge_index_mp0, edge_index_mp1):
        src = ei[0].reshape(NS * NCHUNK, CH)
        dst = ei[1].reshape(NS * NCHUNK, CH)
        edges.append((src, dst))

    perm = np.array(PACK_PERM, np.int32)
    aggs = []
    for li, (F, eler) in enumerate(zip(feats, elers)):
        src, dst = edges[li % 2]
        elerT = jnp.transpose(eler)
        elT = elerT[:HEADS]
        erT = elerT[HEADS:]
        Fb = F[:, :, perm].astype(jnp.bfloat16)
        Fi = lax.bitcast_convert_type(
            Fb.reshape(HEADS, N, HID // 2, 2), jnp.int32)
        out = _gat_sc(src, dst, elT, erT, Fi[0], Fi[1], Fi[2], Fi[3])
        aggs.append(out.reshape(HEADS, N, HID))

    wsum = _run_sem(aggs, params0, params1)
    w = wsum[0] / float(N)
    beta = jnp.concatenate([jax.nn.softmax(w[:2]), jax.nn.softmax(w[2:])])
    beta = beta.reshape(1, 4)

    return _run_head(aggs, beta, params0, params1)


# speedup vs baseline: 36.3490x; 1.0165x over previous
import jax
import jax.numpy as jnp
import numpy as np
from jax import lax
from jax.experimental import pallas as pl
from jax.experimental.pallas import tpu as pltpu
from jax.experimental.pallas import tpu_sc as plsc

N = 10000
E = 160000
HEADS = 4
HID = 64
DOUT = HEADS * HID
SEMD = 128
LABELS = 8

NC = 2
NS = 16
L = 16

EPT = E // NS
CH = 80
NCHUNK = EPT // CH
VPC = CH // L
NDN = 10240
ROWS_PT = N // NS
WBR = 624

BN = 1000
GRID = N // BN

MASKHI = -65536
PACK_PERM = tuple(c for w in range(HID // 2) for c in (w, w + HID // 2))


def _prep_body(h_ref, d0, d1, a0, a1, wd, bdr, wa, bar,
               w00, w01, w10, w11, as0, as1, as2, as3,
               F0, F1, F2, F3, E0, E1, E2, E3):
    mD = (d0[...] + d1[...]) * 0.5
    dD = jnp.dot(mD, wd[...], preferred_element_type=jnp.float32) + bdr[...]
    mA = (a0[...] + a1[...]) * 0.5
    dA = jnp.dot(mA, wa[...], preferred_element_type=jnp.float32) + bar[...]
    h1 = jnp.concatenate([dD, dA], axis=1)
    hh = h_ref[...]
    for F_out, E_out, x, W, As in ((F0, E0, hh, w00, as0),
                                   (F1, E1, hh, w01, as1),
                                   (F2, E2, h1, w10, as2),
                                   (F3, E3, h1, w11, as3)):
        F = jnp.dot(x, W[...], preferred_element_type=jnp.float32)
        for hd in range(HEADS):
            F_out[hd] = F[:, hd * HID:(hd + 1) * HID]
        E_out[...] = jnp.dot(F, As[...], preferred_element_type=jnp.float32)


def _run_prep(h, aD0, aD1, aA0, aA1, Wd, bd, Wa, ba, Ws_list, As_list):
    full = lambda shape: pl.BlockSpec(shape, lambda i: (0,) * len(shape))
    row = lambda shape: pl.BlockSpec(shape, lambda i: (i,) + (0,) * (len(shape) - 1))
    fout = pl.BlockSpec((HEADS, BN, HID), lambda i: (0, i, 0))
    in_specs = [row((BN, 128))] + [row((BN, 64))] * 4 + \
        [full((64, 64)), full((1, 64)), full((64, 64)), full((1, 64))] + \
        [full((128, DOUT))] * 4 + [full((DOUT, 8))] * 4
    out_specs = [fout] * 4 + [row((BN, 8))] * 4
    out_shape = [jax.ShapeDtypeStruct((HEADS, N, HID), jnp.float32)] * 4 + \
                [jax.ShapeDtypeStruct((N, 8), jnp.float32)] * 4
    return pl.pallas_call(
        _prep_body,
        grid=(GRID,),
        in_specs=in_specs,
        out_specs=out_specs,
        out_shape=out_shape,
    )(h, aD0, aD1, aA0, aA1, Wd, bd.reshape(1, 64), Wa, ba.reshape(1, 64),
      *Ws_list, *As_list)


def _gat_sc_body(srcr, dstr, elr, err, f0, f1, f2, f3, outr,
                 src2, dst2, el_v, er_v, exC, rows, rows_i, zdn,
                 acc, sdn):
    c = lax.axis_index("c")
    s = lax.axis_index("s")
    zero16 = jnp.zeros((L,), jnp.float32)

    pltpu.sync_copy(srcr.at[pl.ds(s * NCHUNK, NCHUNK)], src2)
    pltpu.sync_copy(dstr.at[pl.ds(s * NCHUNK, NCHUNK)], dst2)
    pltpu.sync_copy(elr.at[pl.ds(2 * c, 2)], el_v)
    pltpu.sync_copy(err.at[pl.ds(2 * c, 2)], er_v)

    def _zrow(r, carry):
        for k in range(HID // L):
            rows[r, pl.ds(k * L, L)] = zero16
        return carry

    r0 = s * ROWS_PT

    def _zero_acc_dn():
        lax.fori_loop(0, CH, _zrow, 0)
        for q in range(ROWS_PT // CH):
            pltpu.sync_copy(rows, acc.at[pl.ds(r0 + q * CH, CH)])
        rem = ROWS_PT % CH
        pltpu.sync_copy(rows.at[pl.ds(0, rem)],
                        acc.at[pl.ds(r0 + ROWS_PT - rem, rem)])

        def _zdn(i, carry):
            zdn[pl.ds(i * L, L)] = zero16
            return carry
        lax.fori_loop(0, (NDN // NS) // L, _zdn, 0)
        pltpu.sync_copy(zdn, sdn.at[pl.ds(s * (NDN // NS), NDN // NS)])

    _zero_acc_dn()
    plsc.subcore_barrier()

    for hl in range(2):
        hvec = jnp.full((L,), hl, jnp.int32)

        def _phaseA(j, carry):
            for m in range(VPC):
                sv = src2[j, pl.ds(m * L, L)]
                dv = dst2[j, pl.ds(m * L, L)]
                elg = plsc.load_gather(el_v, [hvec, sv])
                erg = plsc.load_gather(er_v, [hvec, dv])
                e = elg + erg
                e = jnp.where(e >= 0.0, e, 0.2 * e)
                exC[pl.ds(j * CH + m * L, L)] = jnp.exp(e)
            return carry
        lax.fori_loop(0, NCHUNK, _phaseA, 0)

        def _dstream(j, carry):
            pltpu.sync_copy(exC.at[pl.ds(j * CH, CH)],
                            sdn.at[dst2.at[j]], add=True)
            return carry
        lax.fori_loop(0, NCHUNK, _dstream, 0)

        plsc.subcore_barrier()

        def _phaseC_one(f):
            def _chunk(j, carry):
                pltpu.sync_copy(f.at[src2.at[j]], rows_i)

                def _scale(m, inner):
                    av = exC[pl.ds(j * CH + m * L, L)]
                    for r16 in range(L):
                        r = m * L + r16
                        a = av[r16]
                        w0 = rows_i[r, pl.ds(0, L)]
                        w1 = rows_i[r, pl.ds(L, L)]
                        rows[r, pl.ds(0, L)] = lax.bitcast_convert_type(
                            jnp.left_shift(w0, 16), jnp.float32) * a
                        rows[r, pl.ds(2 * L, L)] = lax.bitcast_convert_type(
                            jnp.bitwise_and(w0, MASKHI), jnp.float32) * a
                        rows[r, pl.ds(L, L)] = lax.bitcast_convert_type(
                            jnp.left_shift(w1, 16), jnp.float32) * a
                        rows[r, pl.ds(3 * L, L)] = lax.bitcast_convert_type(
                            jnp.bitwise_and(w1, MASKHI), jnp.float32) * a
                    return inner
                lax.fori_loop(0, VPC, _scale, 0)
                pltpu.sync_copy(rows, acc.at[dst2.at[j]], add=True)
                return carry
            lax.fori_loop(0, NCHUNK, _chunk, 0)

        fa, fb = (f0, f2) if hl == 0 else (f1, f3)

        @pl.when(c == 0)
        def _():
            _phaseC_one(fa)

        @pl.when(c == 1)
        def _():
            _phaseC_one(fb)

        plsc.subcore_barrier()

        wbs = s * WBR
        pltpu.sync_copy(sdn.at[pl.ds(wbs, NDN // NS)], zdn)
        base_out = (2 * c + hl) * N + wbs

        def _scale_rows(qoff, nrow):
            for m in range((nrow + L - 1) // L):
                cnt = min(L, nrow - m * L)
                dv = zdn[pl.ds(qoff + m * L, L)]
                iv = 1.0 / (dv + 1e-9)
                for r16 in range(cnt):
                    r = m * L + r16
                    a = iv[r16]
                    for k in range(HID // L):
                        rows[r, pl.ds(k * L, L)] = rows[r, pl.ds(k * L, L)] * a

        NFULL = WBR // CH
        REM = WBR % CH

        def _wb_chunk(q, carry):
            pltpu.sync_copy(acc.at[pl.ds(wbs + q * CH, CH)], rows)
            _scale_rows(q * CH, CH)
            pltpu.sync_copy(rows, outr.at[pl.ds(base_out + q * CH, CH)])
            return carry
        lax.fori_loop(0, NFULL, _wb_chunk, 0)
        pltpu.sync_copy(acc.at[pl.ds(wbs + NFULL * CH, REM)],
                        rows.at[pl.ds(0, REM)])
        _scale_rows(NFULL * CH, REM)
        pltpu.sync_copy(rows.at[pl.ds(0, REM)],
                        outr.at[pl.ds(base_out + NFULL * CH, REM)])

        @pl.when(s == NS - 1)
        def _():
            pltpu.sync_copy(acc.at[pl.ds(wbs + WBR, L)], rows.at[pl.ds(0, L)])
            _scale_rows(WBR, L)
            pltpu.sync_copy(rows.at[pl.ds(0, L)],
                            outr.at[pl.ds(base_out + WBR, L)])

        plsc.subcore_barrier()
        if hl == 0:
            _zero_acc_dn()
            plsc.subcore_barrier()


_gat_sc = pl.kernel(
    _gat_sc_body,
    out_type=jax.ShapeDtypeStruct((HEADS * N, HID), jnp.float32),
    mesh=plsc.VectorSubcoreMesh(core_axis_name="c", subcore_axis_name="s"),
    scratch_types=[
        pltpu.VMEM((NCHUNK, CH), jnp.int32),
        pltpu.VMEM((NCHUNK, CH), jnp.int32),
        pltpu.VMEM((2, N), jnp.float32),
        pltpu.VMEM((2, N), jnp.float32),
        pltpu.VMEM((EPT,), jnp.float32),
        pltpu.VMEM((CH, HID), jnp.float32),
        pltpu.VMEM((CH, HID // 2), jnp.int32),
        pltpu.VMEM((NDN // NS,), jnp.float32),
        pltpu.VMEM_SHARED((N, HID), jnp.float32),
        pltpu.VMEM_SHARED((NDN,), jnp.float32),
    ],
    compiler_params=pltpu.CompilerParams(use_tc_tiling_on_sc=False,
                                         needs_layout_passes=False),
)


def _assemble(ar):
    z = jnp.concatenate([ar[hd] for hd in range(HEADS)], axis=1)
    return jnp.where(z > 0, z, jnp.exp(z) - 1.0)


def _sem_body(a0r, a1r, a2r, a3r, ws0, bs0, qs0, ws1, bs1, qs1, wref):
    i = pl.program_id(0)
    parts = []
    for ar, ws, bs, qs in ((a0r, ws0, bs0, qs0), (a1r, ws0, bs0, qs0),
                           (a2r, ws1, bs1, qs1), (a3r, ws1, bs1, qs1)):
        z = _assemble(ar)
        t = jnp.tanh(jnp.dot(z, ws[...], preferred_element_type=jnp.float32)
                     + bs[...])
        u = jnp.dot(t, qs[...], preferred_element_type=jnp.float32)
        parts.append(jnp.sum(u).reshape(1, 1))
    wvec = jnp.concatenate(parts, axis=1)

    @pl.when(i == 0)
    def _():
        wref[...] = wvec

    @pl.when(i != 0)
    def _():
        wref[...] = wref[...] + wvec


def _run_sem(aggs, p0, p1):
    full = lambda shape: pl.BlockSpec(shape, lambda i: (0,) * len(shape))
    ain = pl.BlockSpec((HEADS, BN, HID), lambda i: (0, i, 0))
    return pl.pallas_call(
        _sem_body,
        grid=(GRID,),
        in_specs=[ain] * 4 + [full((DOUT, SEMD)), full((1, SEMD)),
                              full((SEMD, 1))] * 2,
        out_specs=pl.BlockSpec((1, 4), lambda i: (0, 0)),
        out_shape=jax.ShapeDtypeStruct((1, 4), jnp.float32),
    )(*aggs, p0['Ws'], p0['bs'].reshape(1, SEMD), p0['qs'].reshape(SEMD, 1),
      p1['Ws'], p1['bs'].reshape(1, SEMD), p1['qs'].reshape(SEMD, 1))


def _head_body(a0r, a1r, a2r, a3r, br, wp0, bp0, wp1, bp1, outr):
    b = br[...]
    z0 = _assemble(a0r)
    z1 = _assemble(a1r)
    z2 = _assemble(a2r)
    z3 = _assemble(a3r)
    s0 = z0 * b[0:1, 0:1] + z1 * b[0:1, 1:2]
    l0 = jnp.dot(s0, wp0[...], preferred_element_type=jnp.float32) + bp0[...]
    s1 = z2 * b[0:1, 2:3] + z3 * b[0:1, 3:4]
    l1 = jnp.dot(s1, wp1[...], preferred_element_type=jnp.float32) + bp1[...]
    outr[...] = jax.nn.sigmoid(l0 + 0.1 * l1)


def _run_head(aggs, beta, p0, p1):
    full = lambda shape: pl.BlockSpec(shape, lambda i: (0,) * len(shape))
    ain = pl.BlockSpec((HEADS, BN, HID), lambda i: (0, i, 0))
    return pl.pallas_call(
        _head_body,
        grid=(GRID,),
        in_specs=[ain] * 4 + [full((1, 4)), full((DOUT, LABELS)),
                              full((1, LABELS)), full((DOUT, LABELS)),
                              full((1, LABELS))],
        out_specs=pl.BlockSpec((BN, LABELS), lambda i: (i, 0)),
        out_shape=jax.ShapeDtypeStruct((N, LABELS), jnp.float32),
    )(*aggs, beta, p0['Wp'], p0['bp'].reshape(1, LABELS),
      p1['Wp'], p1['bp'].reshape(1, LABELS))


def _attn_select(p, m):
    al = p['al%d' % m]
    ar = p['ar%d' % m]
    A = jnp.zeros((DOUT, 8), jnp.float32)
    for hd in range(HEADS):
        A = A.at[hd * HID:(hd + 1) * HID, hd].set(al[hd])
        A = A.at[hd * HID:(hd + 1) * HID, 4 + hd].set(ar[hd])
    return A


def kernel(h, aug_D_0, aug_D_1, aug_A_0, aug_A_1, edge_index_mp0,
           edge_index_mp1, Wd, bd, Wa, ba, params0, params1):
    As_list = [_attn_select(params0, 0), _attn_select(params0, 1),
               _attn_select(params1, 0), _attn_select(params1, 1)]
    Ws_list = [params0['W0'], params0['W1'], params1['W0'], params1['W1']]

    prep = _run_prep(h, aug_D_0, aug_D_1, aug_A_0, aug_A_1,
                     Wd, bd, Wa, ba, Ws_list, As_list)
    feats = prep[:4]
    elers = prep[4:]

    edges = []
    for ei in (edge_index_mp0, edge_index_mp1):
        src = ei[0].reshape(NS * NCHUNK, CH)
        dst = ei[1].reshape(NS * NCHUNK, CH)
        edges.append((src, dst))

    perm = np.array(PACK_PERM, np.int32)
    aggs = []
    for li, (F, eler) in enumerate(zip(feats, elers)):
        src, dst = edges[li % 2]
        elerT = jnp.transpose(eler)
        elT = elerT[:HEADS]
        erT = elerT[HEADS:]
        Fb = F[:, :, perm].astype(jnp.bfloat16)
        Fi = lax.bitcast_convert_type(
            Fb.reshape(HEADS, N, HID // 2, 2), jnp.int32)
        out = _gat_sc(src, dst, elT, erT, Fi[0], Fi[1], Fi[2], Fi[3])
        aggs.append(out.reshape(HEADS, N, HID))

    wsum = _run_sem(aggs, params0, params1)
    w = wsum[0] / float(N)
    beta = jnp.concatenate([jax.nn.softmax(w[:2]), jax.nn.softmax(w[2:])])
    beta = beta.reshape(1, 4)

    return _run_head(aggs, beta, params0, params1)


# speedup vs baseline: 52.2733x; 1.4381x over previous
import jax
import jax.numpy as jnp
import numpy as np
from jax import lax
from jax.experimental import pallas as pl
from jax.experimental.pallas import tpu as pltpu
from jax.experimental.pallas import tpu_sc as plsc

N = 10000
E = 160000
HEADS = 4
HID = 64
DOUT = HEADS * HID
SEMD = 128
LABELS = 8

NC = 2
NS = 16
L = 16

EPT = E // NS
CH = 80
NCHUNK = EPT // CH
VPC = CH // L
NDN = 10240
ROWS_PT = N // NS
WBR = 624

BN = 1000
GRID = N // BN

MASKHI = -65536
PACK_PERM = tuple(c for w in range(HID // 2) for c in (w, w + HID // 2))


def _prep_body(h_ref, d0, d1, a0, a1, wd, bdr, wa, bar,
               w00, w01, w10, w11, as0, as1, as2, as3,
               F0, F1, F2, F3, E0, E1, E2, E3):
    mD = (d0[...] + d1[...]) * 0.5
    dD = jnp.dot(mD, wd[...], preferred_element_type=jnp.float32) + bdr[...]
    mA = (a0[...] + a1[...]) * 0.5
    dA = jnp.dot(mA, wa[...], preferred_element_type=jnp.float32) + bar[...]
    h1 = jnp.concatenate([dD, dA], axis=1)
    hh = h_ref[...]
    for F_out, E_out, x, W, As in ((F0, E0, hh, w00, as0),
                                   (F1, E1, hh, w01, as1),
                                   (F2, E2, h1, w10, as2),
                                   (F3, E3, h1, w11, as3)):
        F = jnp.dot(x, W[...], preferred_element_type=jnp.float32)
        for hd in range(HEADS):
            F_out[hd] = F[:, hd * HID:(hd + 1) * HID]
        E_out[...] = jnp.dot(F, As[...], preferred_element_type=jnp.float32)


def _run_prep(h, aD0, aD1, aA0, aA1, Wd, bd, Wa, ba, Ws_list, As_list):
    full = lambda shape: pl.BlockSpec(shape, lambda i: (0,) * len(shape))
    row = lambda shape: pl.BlockSpec(shape, lambda i: (i,) + (0,) * (len(shape) - 1))
    fout = pl.BlockSpec((HEADS, BN, HID), lambda i: (0, i, 0))
    in_specs = [row((BN, 128))] + [row((BN, 64))] * 4 + \
        [full((64, 64)), full((1, 64)), full((64, 64)), full((1, 64))] + \
        [full((128, DOUT))] * 4 + [full((DOUT, 8))] * 4
    out_specs = [fout] * 4 + [row((BN, 8))] * 4
    out_shape = [jax.ShapeDtypeStruct((HEADS, N, HID), jnp.float32)] * 4 + \
                [jax.ShapeDtypeStruct((N, 8), jnp.float32)] * 4
    return pl.pallas_call(
        _prep_body,
        grid=(GRID,),
        in_specs=in_specs,
        out_specs=out_specs,
        out_shape=out_shape,
    )(h, aD0, aD1, aA0, aA1, Wd, bd.reshape(1, 64), Wa, ba.reshape(1, 64),
      *Ws_list, *As_list)


def _gat_sc_body(srcr, dstr, elr, err, f0, f1, f2, f3, outr,
                 src2, dst2, el_v, er_v, exC, rows, rows_i, rows_i2, zdn,
                 acc, sdn, sem):
    c = lax.axis_index("c")
    s = lax.axis_index("s")
    zero16 = jnp.zeros((L,), jnp.float32)

    pltpu.sync_copy(srcr.at[pl.ds(s * NCHUNK, NCHUNK)], src2)
    pltpu.sync_copy(dstr.at[pl.ds(s * NCHUNK, NCHUNK)], dst2)
    pltpu.sync_copy(elr.at[pl.ds(2 * c, 2)], el_v)
    pltpu.sync_copy(err.at[pl.ds(2 * c, 2)], er_v)

    def _zrow(r, carry):
        for k in range(HID // L):
            rows[r, pl.ds(k * L, L)] = zero16
        return carry

    r0 = s * ROWS_PT

    def _zero_acc_dn():
        lax.fori_loop(0, CH, _zrow, 0)
        for q in range(ROWS_PT // CH):
            pltpu.sync_copy(rows, acc.at[pl.ds(r0 + q * CH, CH)])
        rem = ROWS_PT % CH
        pltpu.sync_copy(rows.at[pl.ds(0, rem)],
                        acc.at[pl.ds(r0 + ROWS_PT - rem, rem)])

        def _zdn(i, carry):
            zdn[pl.ds(i * L, L)] = zero16
            return carry
        lax.fori_loop(0, (NDN // NS) // L, _zdn, 0)
        pltpu.sync_copy(zdn, sdn.at[pl.ds(s * (NDN // NS), NDN // NS)])

    _zero_acc_dn()
    plsc.subcore_barrier()

    for hl in range(2):
        hvec = jnp.full((L,), hl, jnp.int32)

        def _phaseA(j, carry):
            for m in range(VPC):
                sv = src2[j, pl.ds(m * L, L)]
                dv = dst2[j, pl.ds(m * L, L)]
                elg = plsc.load_gather(el_v, [hvec, sv])
                erg = plsc.load_gather(er_v, [hvec, dv])
                e = elg + erg
                e = jnp.where(e >= 0.0, e, 0.2 * e)
                exC[pl.ds(j * CH + m * L, L)] = jnp.exp(e)
            return carry
        lax.fori_loop(0, NCHUNK, _phaseA, 0)

        def _dstream(j, carry):
            pltpu.sync_copy(exC.at[pl.ds(j * CH, CH)],
                            sdn.at[dst2.at[j]], add=True)
            return carry
        lax.fori_loop(0, NCHUNK, _dstream, 0)

        plsc.subcore_barrier()

        def _proc(j, buf):
            def _scale(m, inner):
                av = exC[pl.ds(j * CH + m * L, L)]
                for r16 in range(L):
                    r = m * L + r16
                    a = av[r16]
                    w0 = buf[r, pl.ds(0, L)]
                    w1 = buf[r, pl.ds(L, L)]
                    rows[r, pl.ds(0, L)] = lax.bitcast_convert_type(
                        jnp.left_shift(w0, 16), jnp.float32) * a
                    rows[r, pl.ds(2 * L, L)] = lax.bitcast_convert_type(
                        jnp.bitwise_and(w0, MASKHI), jnp.float32) * a
                    rows[r, pl.ds(L, L)] = lax.bitcast_convert_type(
                        jnp.left_shift(w1, 16), jnp.float32) * a
                    rows[r, pl.ds(3 * L, L)] = lax.bitcast_convert_type(
                        jnp.bitwise_and(w1, MASKHI), jnp.float32) * a
                return inner
            lax.fori_loop(0, VPC, _scale, 0)
            pltpu.sync_copy(rows, acc.at[dst2.at[j]], add=True)

        def _phaseC_one(f):
            pltpu.async_copy(f.at[src2.at[0]], rows_i, sem)

            def _pair(p, carry):
                j0 = 2 * p
                pltpu.make_async_copy(f.at[src2.at[j0]], rows_i, sem).wait()
                pltpu.async_copy(f.at[src2.at[j0 + 1]], rows_i2, sem)
                _proc(j0, rows_i)
                pltpu.make_async_copy(
                    f.at[src2.at[j0 + 1]], rows_i2, sem).wait()
                pltpu.async_copy(f.at[src2.at[j0 + 2]], rows_i, sem)
                _proc(j0 + 1, rows_i2)
                return carry
            lax.fori_loop(0, (NCHUNK - 1) // 2, _pair, 0)
            pltpu.make_async_copy(
                f.at[src2.at[NCHUNK - 1]], rows_i, sem).wait()
            _proc(NCHUNK - 1, rows_i)

        fa, fb = (f0, f2) if hl == 0 else (f1, f3)

        @pl.when(c == 0)
        def _():
            _phaseC_one(fa)

        @pl.when(c == 1)
        def _():
            _phaseC_one(fb)

        plsc.subcore_barrier()

        wbs = s * WBR
        pltpu.sync_copy(sdn.at[pl.ds(wbs, NDN // NS)], zdn)
        base_out = (2 * c + hl) * N + wbs

        def _scale_rows(qoff, nrow):
            for m in range((nrow + L - 1) // L):
                cnt = min(L, nrow - m * L)
                dv = zdn[pl.ds(qoff + m * L, L)]
                iv = 1.0 / (dv + 1e-9)
                for r16 in range(cnt):
                    r = m * L + r16
                    a = iv[r16]
                    for k in range(HID // L):
                        rows[r, pl.ds(k * L, L)] = rows[r, pl.ds(k * L, L)] * a

        NFULL = WBR // CH
        REM = WBR % CH

        def _wb_chunk(q, carry):
            pltpu.sync_copy(acc.at[pl.ds(wbs + q * CH, CH)], rows)
            _scale_rows(q * CH, CH)
            pltpu.sync_copy(rows, outr.at[pl.ds(base_out + q * CH, CH)])
            return carry
        lax.fori_loop(0, NFULL, _wb_chunk, 0)
        pltpu.sync_copy(acc.at[pl.ds(wbs + NFULL * CH, REM)],
                        rows.at[pl.ds(0, REM)])
        _scale_rows(NFULL * CH, REM)
        pltpu.sync_copy(rows.at[pl.ds(0, REM)],
                        outr.at[pl.ds(base_out + NFULL * CH, REM)])

        @pl.when(s == NS - 1)
        def _():
            pltpu.sync_copy(acc.at[pl.ds(wbs + WBR, L)], rows.at[pl.ds(0, L)])
            _scale_rows(WBR, L)
            pltpu.sync_copy(rows.at[pl.ds(0, L)],
                            outr.at[pl.ds(base_out + WBR, L)])

        plsc.subcore_barrier()
        if hl == 0:
            _zero_acc_dn()
            plsc.subcore_barrier()


_gat_sc = pl.kernel(
    _gat_sc_body,
    out_type=jax.ShapeDtypeStruct((HEADS * N, HID), jnp.float32),
    mesh=plsc.VectorSubcoreMesh(core_axis_name="c", subcore_axis_name="s"),
    scratch_types=[
        pltpu.VMEM((NCHUNK, CH), jnp.int32),
        pltpu.VMEM((NCHUNK, CH), jnp.int32),
        pltpu.VMEM((2, N), jnp.float32),
        pltpu.VMEM((2, N), jnp.float32),
        pltpu.VMEM((EPT,), jnp.float32),
        pltpu.VMEM((CH, HID), jnp.float32),
        pltpu.VMEM((CH, HID // 2), jnp.int32),
        pltpu.VMEM((CH, HID // 2), jnp.int32),
        pltpu.VMEM((NDN // NS,), jnp.float32),
        pltpu.VMEM_SHARED((N, HID), jnp.float32),
        pltpu.VMEM_SHARED((NDN,), jnp.float32),
        pltpu.SemaphoreType.DMA,
    ],
    compiler_params=pltpu.CompilerParams(use_tc_tiling_on_sc=False,
                                         needs_layout_passes=False),
)


def _assemble(ar):
    z = jnp.concatenate([ar[hd] for hd in range(HEADS)], axis=1)
    return jnp.where(z > 0, z, jnp.exp(z) - 1.0)


def _sem_body(a0r, a1r, a2r, a3r, ws0, bs0, qs0, ws1, bs1, qs1, wref):
    i = pl.program_id(0)
    parts = []
    for ar, ws, bs, qs in ((a0r, ws0, bs0, qs0), (a1r, ws0, bs0, qs0),
                           (a2r, ws1, bs1, qs1), (a3r, ws1, bs1, qs1)):
        z = _assemble(ar)
        t = jnp.tanh(jnp.dot(z, ws[...], preferred_element_type=jnp.float32)
                     + bs[...])
        u = jnp.dot(t, qs[...], preferred_element_type=jnp.float32)
        parts.append(jnp.sum(u).reshape(1, 1))
    wvec = jnp.concatenate(parts, axis=1)

    @pl.when(i == 0)
    def _():
        wref[...] = wvec

    @pl.when(i != 0)
    def _():
        wref[...] = wref[...] + wvec


def _run_sem(aggs, p0, p1):
    full = lambda shape: pl.BlockSpec(shape, lambda i: (0,) * len(shape))
    ain = pl.BlockSpec((HEADS, BN, HID), lambda i: (0, i, 0))
    return pl.pallas_call(
        _sem_body,
        grid=(GRID,),
        in_specs=[ain] * 4 + [full((DOUT, SEMD)), full((1, SEMD)),
                              full((SEMD, 1))] * 2,
        out_specs=pl.BlockSpec((1, 4), lambda i: (0, 0)),
        out_shape=jax.ShapeDtypeStruct((1, 4), jnp.float32),
    )(*aggs, p0['Ws'], p0['bs'].reshape(1, SEMD), p0['qs'].reshape(SEMD, 1),
      p1['Ws'], p1['bs'].reshape(1, SEMD), p1['qs'].reshape(SEMD, 1))


def _head_body(a0r, a1r, a2r, a3r, br, wp0, bp0, wp1, bp1, outr):
    b = br[...]
    z0 = _assemble(a0r)
    z1 = _assemble(a1r)
    z2 = _assemble(a2r)
    z3 = _assemble(a3r)
    s0 = z0 * b[0:1, 0:1] + z1 * b[0:1, 1:2]
    l0 = jnp.dot(s0, wp0[...], preferred_element_type=jnp.float32) + bp0[...]
    s1 = z2 * b[0:1, 2:3] + z3 * b[0:1, 3:4]
    l1 = jnp.dot(s1, wp1[...], preferred_element_type=jnp.float32) + bp1[...]
    outr[...] = jax.nn.sigmoid(l0 + 0.1 * l1)


def _run_head(aggs, beta, p0, p1):
    full = lambda shape: pl.BlockSpec(shape, lambda i: (0,) * len(shape))
    ain = pl.BlockSpec((HEADS, BN, HID), lambda i: (0, i, 0))
    return pl.pallas_call(
        _head_body,
        grid=(GRID,),
        in_specs=[ain] * 4 + [full((1, 4)), full((DOUT, LABELS)),
                              full((1, LABELS)), full((DOUT, LABELS)),
                              full((1, LABELS))],
        out_specs=pl.BlockSpec((BN, LABELS), lambda i: (i, 0)),
        out_shape=jax.ShapeDtypeStruct((N, LABELS), jnp.float32),
    )(*aggs, beta, p0['Wp'], p0['bp'].reshape(1, LABELS),
      p1['Wp'], p1['bp'].reshape(1, LABELS))


def _attn_select(p, m):
    al = p['al%d' % m]
    ar = p['ar%d' % m]
    A = jnp.zeros((DOUT, 8), jnp.float32)
    for hd in range(HEADS):
        A = A.at[hd * HID:(hd + 1) * HID, hd].set(al[hd])
        A = A.at[hd * HID:(hd + 1) * HID, 4 + hd].set(ar[hd])
    return A


def kernel(h, aug_D_0, aug_D_1, aug_A_0, aug_A_1, edge_index_mp0,
           edge_index_mp1, Wd, bd, Wa, ba, params0, params1):
    As_list = [_attn_select(params0, 0), _attn_select(params0, 1),
               _attn_select(params1, 0), _attn_select(params1, 1)]
    Ws_list = [params0['W0'], params0['W1'], params1['W0'], params1['W1']]

    prep = _run_prep(h, aug_D_0, aug_D_1, aug_A_0, aug_A_1,
                     Wd, bd, Wa, ba, Ws_list, As_list)
    feats = prep[:4]
    elers = prep[4:]

    edges = []
    for ei in (edge_index_mp0, edge_index_mp1):
        src = ei[0].reshape(NS * NCHUNK, CH)
        dst = ei[1].reshape(NS * NCHUNK, CH)
        edges.append((src, dst))

    perm = np.array(PACK_PERM, np.int32)
    aggs = []
    for li, (F, eler) in enumerate(zip(feats, elers)):
        src, dst = edges[li % 2]
        elerT = jnp.transpose(eler)
        elT = elerT[:HEADS]
        erT = elerT[HEADS:]
        Fb = F[:, :, perm].astype(jnp.bfloat16)
        Fi = lax.bitcast_convert_type(
            Fb.reshape(HEADS, N, HID // 2, 2), jnp.int32)
        out = _gat_sc(src, dst, elT, erT, Fi[0], Fi[1], Fi[2], Fi[3])
        aggs.append(out.reshape(HEADS, N, HID))

    wsum = _run_sem(aggs, params0, params1)
    w = wsum[0] / float(N)
    beta = jnp.concatenate([jax.nn.softmax(w[:2]), jax.nn.softmax(w[2:])])
    beta = beta.reshape(1, 4)

    return _run_head(aggs, beta, params0, params1)


# speedup vs baseline: 55.6414x; 1.0644x over previous
import jax
import jax.numpy as jnp
import numpy as np
from jax import lax
from jax.experimental import pallas as pl
from jax.experimental.pallas import tpu as pltpu
from jax.experimental.pallas import tpu_sc as plsc

N = 10000
E = 160000
HEADS = 4
HID = 64
DOUT = HEADS * HID
SEMD = 128
LABELS = 8

NC = 2
NS = 16
L = 16

EPT = E // NS
CH = 80
NCHUNK = EPT // CH
VPC = CH // L
NDN = 10240
ROWS_PT = N // NS
WBR = 624

BN = 1000
GRID = N // BN

MASKHI = -65536
PACK_PERM = tuple(c for w in range(HID // 2) for c in (w, w + HID // 2))


def _prep_body(h_ref, d0, d1, a0, a1, wd, bdr, wa, bar,
               w00, w01, w10, w11, as0, as1, as2, as3,
               F0, F1, F2, F3, E0, E1, E2, E3):
    mD = (d0[...] + d1[...]) * 0.5
    dD = jnp.dot(mD, wd[...], preferred_element_type=jnp.float32) + bdr[...]
    mA = (a0[...] + a1[...]) * 0.5
    dA = jnp.dot(mA, wa[...], preferred_element_type=jnp.float32) + bar[...]
    h1 = jnp.concatenate([dD, dA], axis=1)
    hh = h_ref[...]
    for F_out, E_out, x, W, As in ((F0, E0, hh, w00, as0),
                                   (F1, E1, hh, w01, as1),
                                   (F2, E2, h1, w10, as2),
                                   (F3, E3, h1, w11, as3)):
        F = jnp.dot(x, W[...], preferred_element_type=jnp.float32)
        for hd in range(HEADS):
            F_out[hd] = F[:, hd * HID:(hd + 1) * HID]
        E_out[...] = jnp.dot(F, As[...], preferred_element_type=jnp.float32)


def _run_prep(h, aD0, aD1, aA0, aA1, Wd, bd, Wa, ba, Ws_list, As_list):
    full = lambda shape: pl.BlockSpec(shape, lambda i: (0,) * len(shape))
    row = lambda shape: pl.BlockSpec(shape, lambda i: (i,) + (0,) * (len(shape) - 1))
    fout = pl.BlockSpec((HEADS, BN, HID), lambda i: (0, i, 0))
    in_specs = [row((BN, 128))] + [row((BN, 64))] * 4 + \
        [full((64, 64)), full((1, 64)), full((64, 64)), full((1, 64))] + \
        [full((128, DOUT))] * 4 + [full((DOUT, 8))] * 4
    out_specs = [fout] * 4 + [row((BN, 8))] * 4
    out_shape = [jax.ShapeDtypeStruct((HEADS, N, HID), jnp.float32)] * 4 + \
                [jax.ShapeDtypeStruct((N, 8), jnp.float32)] * 4
    return pl.pallas_call(
        _prep_body,
        grid=(GRID,),
        in_specs=in_specs,
        out_specs=out_specs,
        out_shape=out_shape,
    )(h, aD0, aD1, aA0, aA1, Wd, bd.reshape(1, 64), Wa, ba.reshape(1, 64),
      *Ws_list, *As_list)


def _gat_sc_body(srcr, dstr, elr, err, f0, f1, f2, f3, outr,
                 src2, dst2, el_v, er_v, exC, rows, rows2, rows_i, rows_i2,
                 zdn, acc, sdn, sem, semE, semO):
    c = lax.axis_index("c")
    s = lax.axis_index("s")
    zero16 = jnp.zeros((L,), jnp.float32)

    pltpu.sync_copy(srcr.at[pl.ds(s * NCHUNK, NCHUNK)], src2)
    pltpu.sync_copy(dstr.at[pl.ds(s * NCHUNK, NCHUNK)], dst2)
    pltpu.sync_copy(elr.at[pl.ds(2 * c, 2)], el_v)
    pltpu.sync_copy(err.at[pl.ds(2 * c, 2)], er_v)

    def _zrow(r, carry):
        for k in range(HID // L):
            rows[r, pl.ds(k * L, L)] = zero16
        return carry

    r0 = s * ROWS_PT

    def _zero_acc_dn():
        lax.fori_loop(0, CH, _zrow, 0)
        for q in range(ROWS_PT // CH):
            pltpu.sync_copy(rows, acc.at[pl.ds(r0 + q * CH, CH)])
        rem = ROWS_PT % CH
        pltpu.sync_copy(rows.at[pl.ds(0, rem)],
                        acc.at[pl.ds(r0 + ROWS_PT - rem, rem)])

        def _zdn(i, carry):
            zdn[pl.ds(i * L, L)] = zero16
            return carry
        lax.fori_loop(0, (NDN // NS) // L, _zdn, 0)
        pltpu.sync_copy(zdn, sdn.at[pl.ds(s * (NDN // NS), NDN // NS)])

    _zero_acc_dn()
    plsc.subcore_barrier()

    for hl in range(2):
        hvec = jnp.full((L,), hl, jnp.int32)

        def _phaseA(j, carry):
            for m in range(VPC):
                sv = src2[j, pl.ds(m * L, L)]
                dv = dst2[j, pl.ds(m * L, L)]
                elg = plsc.load_gather(el_v, [hvec, sv])
                erg = plsc.load_gather(er_v, [hvec, dv])
                e = elg + erg
                e = jnp.where(e >= 0.0, e, 0.2 * e)
                exC[pl.ds(j * CH + m * L, L)] = jnp.exp(e)
            return carry
        lax.fori_loop(0, NCHUNK, _phaseA, 0)

        def _dstream(j, carry):
            pltpu.sync_copy(exC.at[pl.ds(j * CH, CH)],
                            sdn.at[dst2.at[j]], add=True)
            return carry
        lax.fori_loop(0, NCHUNK, _dstream, 0)

        plsc.subcore_barrier()

        def _scale(j, buf, rbuf):
            def _body(m, inner):
                av = exC[pl.ds(j * CH + m * L, L)]
                for r16 in range(L):
                    r = m * L + r16
                    a = av[r16]
                    w0 = buf[r, pl.ds(0, L)]
                    w1 = buf[r, pl.ds(L, L)]
                    rbuf[r, pl.ds(0, L)] = lax.bitcast_convert_type(
                        jnp.left_shift(w0, 16), jnp.float32) * a
                    rbuf[r, pl.ds(2 * L, L)] = lax.bitcast_convert_type(
                        jnp.bitwise_and(w0, MASKHI), jnp.float32) * a
                    rbuf[r, pl.ds(L, L)] = lax.bitcast_convert_type(
                        jnp.left_shift(w1, 16), jnp.float32) * a
                    rbuf[r, pl.ds(3 * L, L)] = lax.bitcast_convert_type(
                        jnp.bitwise_and(w1, MASKHI), jnp.float32) * a
                return inner
            lax.fori_loop(0, VPC, _body, 0)

        def _wait_scatter(rbuf, rsem):
            pltpu.make_async_copy(outr.at[pl.ds(0, CH)], rbuf, rsem).wait()

        def _phaseC_one(f):
            pltpu.async_copy(f.at[src2.at[0]], rows_i, sem)

            def _pair(p, carry):
                j0 = 2 * p
                pltpu.make_async_copy(f.at[src2.at[j0]], rows_i, sem).wait()
                pltpu.async_copy(f.at[src2.at[j0 + 1]], rows_i2, sem)

                @pl.when(j0 >= 2)
                def _():
                    _wait_scatter(rows, semE)
                _scale(j0, rows_i, rows)
                pltpu.async_copy(rows, acc.at[dst2.at[j0]], semE, add=True)

                pltpu.make_async_copy(
                    f.at[src2.at[j0 + 1]], rows_i2, sem).wait()
                pltpu.async_copy(f.at[src2.at[j0 + 2]], rows_i, sem)

                @pl.when(j0 >= 2)
                def _():
                    _wait_scatter(rows2, semO)
                _scale(j0 + 1, rows_i2, rows2)
                pltpu.async_copy(rows2, acc.at[dst2.at[j0 + 1]], semO,
                                 add=True)
                return carry
            lax.fori_loop(0, (NCHUNK - 1) // 2, _pair, 0)

            pltpu.make_async_copy(
                f.at[src2.at[NCHUNK - 1]], rows_i, sem).wait()
            _wait_scatter(rows, semE)
            _scale(NCHUNK - 1, rows_i, rows)
            pltpu.async_copy(rows, acc.at[dst2.at[NCHUNK - 1]], semE,
                             add=True)
            _wait_scatter(rows, semE)
            _wait_scatter(rows2, semO)

        fa, fb = (f0, f2) if hl == 0 else (f1, f3)

        @pl.when(c == 0)
        def _():
            _phaseC_one(fa)

        @pl.when(c == 1)
        def _():
            _phaseC_one(fb)

        plsc.subcore_barrier()

        wbs = s * WBR
        pltpu.sync_copy(sdn.at[pl.ds(wbs, NDN // NS)], zdn)
        base_out = (2 * c + hl) * N + wbs

        def _scale_rows(qoff, nrow):
            for m in range((nrow + L - 1) // L):
                cnt = min(L, nrow - m * L)
                dv = zdn[pl.ds(qoff + m * L, L)]
                iv = 1.0 / (dv + 1e-9)
                for r16 in range(cnt):
                    r = m * L + r16
                    a = iv[r16]
                    for k in range(HID // L):
                        rows[r, pl.ds(k * L, L)] = rows[r, pl.ds(k * L, L)] * a

        NFULL = WBR // CH
        REM = WBR % CH

        def _wb_chunk(q, carry):
            pltpu.sync_copy(acc.at[pl.ds(wbs + q * CH, CH)], rows)
            _scale_rows(q * CH, CH)
            pltpu.sync_copy(rows, outr.at[pl.ds(base_out + q * CH, CH)])
            return carry
        lax.fori_loop(0, NFULL, _wb_chunk, 0)
        pltpu.sync_copy(acc.at[pl.ds(wbs + NFULL * CH, REM)],
                        rows.at[pl.ds(0, REM)])
        _scale_rows(NFULL * CH, REM)
        pltpu.sync_copy(rows.at[pl.ds(0, REM)],
                        outr.at[pl.ds(base_out + NFULL * CH, REM)])

        @pl.when(s == NS - 1)
        def _():
            pltpu.sync_copy(acc.at[pl.ds(wbs + WBR, L)], rows.at[pl.ds(0, L)])
            _scale_rows(WBR, L)
            pltpu.sync_copy(rows.at[pl.ds(0, L)],
                            outr.at[pl.ds(base_out + WBR, L)])

        plsc.subcore_barrier()
        if hl == 0:
            _zero_acc_dn()
            plsc.subcore_barrier()


_gat_sc = pl.kernel(
    _gat_sc_body,
    out_type=jax.ShapeDtypeStruct((HEADS * N, HID), jnp.float32),
    mesh=plsc.VectorSubcoreMesh(core_axis_name="c", subcore_axis_name="s"),
    scratch_types=[
        pltpu.VMEM((NCHUNK, CH), jnp.int32),
        pltpu.VMEM((NCHUNK, CH), jnp.int32),
        pltpu.VMEM((2, N), jnp.float32),
        pltpu.VMEM((2, N), jnp.float32),
        pltpu.VMEM((EPT,), jnp.float32),
        pltpu.VMEM((CH, HID), jnp.float32),
        pltpu.VMEM((CH, HID), jnp.float32),
        pltpu.VMEM((CH, HID // 2), jnp.int32),
        pltpu.VMEM((CH, HID // 2), jnp.int32),
        pltpu.VMEM((NDN // NS,), jnp.float32),
        pltpu.VMEM_SHARED((N, HID), jnp.float32),
        pltpu.VMEM_SHARED((NDN,), jnp.float32),
        pltpu.SemaphoreType.DMA,
        pltpu.SemaphoreType.DMA,
        pltpu.SemaphoreType.DMA,
    ],
    compiler_params=pltpu.CompilerParams(use_tc_tiling_on_sc=False,
                                         needs_layout_passes=False),
)


def _assemble(ar):
    z = jnp.concatenate([ar[hd] for hd in range(HEADS)], axis=1)
    return jnp.where(z > 0, z, jnp.exp(z) - 1.0)


def _sem_body(a0r, a1r, a2r, a3r, ws0, bs0, qs0, ws1, bs1, qs1, wref):
    i = pl.program_id(0)
    parts = []
    for ar, ws, bs, qs in ((a0r, ws0, bs0, qs0), (a1r, ws0, bs0, qs0),
                           (a2r, ws1, bs1, qs1), (a3r, ws1, bs1, qs1)):
        z = _assemble(ar)
        t = jnp.tanh(jnp.dot(z, ws[...], preferred_element_type=jnp.float32)
                     + bs[...])
        u = jnp.dot(t, qs[...], preferred_element_type=jnp.float32)
        parts.append(jnp.sum(u).reshape(1, 1))
    wvec = jnp.concatenate(parts, axis=1)

    @pl.when(i == 0)
    def _():
        wref[...] = wvec

    @pl.when(i != 0)
    def _():
        wref[...] = wref[...] + wvec


def _run_sem(aggs, p0, p1):
    full = lambda shape: pl.BlockSpec(shape, lambda i: (0,) * len(shape))
    ain = pl.BlockSpec((HEADS, BN, HID), lambda i: (0, i, 0))
    return pl.pallas_call(
        _sem_body,
        grid=(GRID,),
        in_specs=[ain] * 4 + [full((DOUT, SEMD)), full((1, SEMD)),
                              full((SEMD, 1))] * 2,
        out_specs=pl.BlockSpec((1, 4), lambda i: (0, 0)),
        out_shape=jax.ShapeDtypeStruct((1, 4), jnp.float32),
    )(*aggs, p0['Ws'], p0['bs'].reshape(1, SEMD), p0['qs'].reshape(SEMD, 1),
      p1['Ws'], p1['bs'].reshape(1, SEMD), p1['qs'].reshape(SEMD, 1))


def _head_body(a0r, a1r, a2r, a3r, br, wp0, bp0, wp1, bp1, outr):
    b = br[...]
    z0 = _assemble(a0r)
    z1 = _assemble(a1r)
    z2 = _assemble(a2r)
    z3 = _assemble(a3r)
    s0 = z0 * b[0:1, 0:1] + z1 * b[0:1, 1:2]
    l0 = jnp.dot(s0, wp0[...], preferred_element_type=jnp.float32) + bp0[...]
    s1 = z2 * b[0:1, 2:3] + z3 * b[0:1, 3:4]
    l1 = jnp.dot(s1, wp1[...], preferred_element_type=jnp.float32) + bp1[...]
    outr[...] = jax.nn.sigmoid(l0 + 0.1 * l1)


def _run_head(aggs, beta, p0, p1):
    full = lambda shape: pl.BlockSpec(shape, lambda i: (0,) * len(shape))
    ain = pl.BlockSpec((HEADS, BN, HID), lambda i: (0, i, 0))
    return pl.pallas_call(
        _head_body,
        grid=(GRID,),
        in_specs=[ain] * 4 + [full((1, 4)), full((DOUT, LABELS)),
                              full((1, LABELS)), full((DOUT, LABELS)),
                              full((1, LABELS))],
        out_specs=pl.BlockSpec((BN, LABELS), lambda i: (i, 0)),
        out_shape=jax.ShapeDtypeStruct((N, LABELS), jnp.float32),
    )(*aggs, beta, p0['Wp'], p0['bp'].reshape(1, LABELS),
      p1['Wp'], p1['bp'].reshape(1, LABELS))


def _attn_select(p, m):
    al = p['al%d' % m]
    ar = p['ar%d' % m]
    A = jnp.zeros((DOUT, 8), jnp.float32)
    for hd in range(HEADS):
        A = A.at[hd * HID:(hd + 1) * HID, hd].set(al[hd])
        A = A.at[hd * HID:(hd + 1) * HID, 4 + hd].set(ar[hd])
    return A


def kernel(h, aug_D_0, aug_D_1, aug_A_0, aug_A_1, edge_index_mp0,
           edge_index_mp1, Wd, bd, Wa, ba, params0, params1):
    As_list = [_attn_select(params0, 0), _attn_select(params0, 1),
               _attn_select(params1, 0), _attn_select(params1, 1)]
    Ws_list = [params0['W0'], params0['W1'], params1['W0'], params1['W1']]

    prep = _run_prep(h, aug_D_0, aug_D_1, aug_A_0, aug_A_1,
                     Wd, bd, Wa, ba, Ws_list, As_list)
    feats = prep[:4]
    elers = prep[4:]

    edges = []
    for ei in (edge_index_mp0, edge_index_mp1):
        src = ei[0].reshape(NS * NCHUNK, CH)
        dst = ei[1].reshape(NS * NCHUNK, CH)
        edges.append((src, dst))

    perm = np.array(PACK_PERM, np.int32)
    aggs = []
    for li, (F, eler) in enumerate(zip(feats, elers)):
        src, dst = edges[li % 2]
        elerT = jnp.transpose(eler)
        elT = elerT[:HEADS]
        erT = elerT[HEADS:]
        Fb = F[:, :, perm].astype(jnp.bfloat16)
        Fi = lax.bitcast_convert_type(
            Fb.reshape(HEADS, N, HID // 2, 2), jnp.int32)
        out = _gat_sc(src, dst, elT, erT, Fi[0], Fi[1], Fi[2], Fi[3])
        aggs.append(out.reshape(HEADS, N, HID))

    wsum = _run_sem(aggs, params0, params1)
    w = wsum[0] / float(N)
    beta = jnp.concatenate([jax.nn.softmax(w[:2]), jax.nn.softmax(w[2:])])
    beta = beta.reshape(1, 4)

    return _run_head(aggs, beta, params0, params1)
